# Initial kernel scaffold; baseline (speedup 1.0000x reference)
import jax
import jax.numpy as jnp
from jax.experimental import pallas as pl


def kernel(x, w_dw, s_dw, w_pw, s_pw):
    raise NotImplementedError("write your pallas kernel here")


# speedup vs baseline: 1.3361x; 1.3361x over previous
import jax
import jax.numpy as jnp
from jax import lax
from jax.experimental import pallas as pl
from jax.experimental.pallas import tpu as pltpu


def _dsconv_kernel(x_ref, wt_ref, sdw_ref, wpt_ref, spw_ref, o_ref, *, W, HW):
    x = x_ref[0]
    C = x.shape[0]

    lane = lax.broadcasted_iota(jnp.int32, (1, HW), 1)
    wi = lax.rem(lane, W)
    mask_w = (wi != 0).astype(jnp.float32)
    mask_e = (wi != W - 1).astype(jnp.float32)
    mask_n = (lane >= W).astype(jnp.float32)
    mask_s = (lane < HW - W).astype(jnp.float32)

    s_west = pltpu.roll(x, 1, axis=1) * mask_w
    s_east = pltpu.roll(x, HW - 1, axis=1) * mask_e
    variants = (s_west, x, s_east)

    acc = jnp.zeros_like(x)
    for dy in range(3):
        shift = (1 - dy) * W
        t = jnp.zeros_like(x)
        for dx in range(3):
            v = variants[dx]
            if shift:
                v = pltpu.roll(v, shift % HW, axis=1)
            t = t + v * wt_ref[:, 3 * dy + dx][:, None]
        if dy == 0:
            t = t * mask_n
        elif dy == 2:
            t = t * mask_s
        acc = acc + t
    dw = jnp.maximum(acc + sdw_ref[:, 0][:, None], 0.0)

    pw = jnp.dot(wpt_ref[...], dw.astype(wpt_ref.dtype),
                 preferred_element_type=jnp.float32)
    o_ref[0] = jnp.maximum(pw + spw_ref[:, 0][:, None], 0.0)


def kernel(x, w_dw, s_dw, w_pw, s_pw):
    N, C, H, W = x.shape
    HW = H * W
    Co = w_pw.shape[1] // W

    taps = w_dw[:, :, C:2 * C]
    wt = jnp.transpose(taps, (2, 0, 1)).reshape(C, 9).astype(jnp.float32)
    sdw = s_dw[0, :C].reshape(C, 1).astype(jnp.float32)
    wpt = jnp.transpose(w_pw[:C, :Co]).astype(jnp.bfloat16)
    spw = s_pw[0, :Co].reshape(Co, 1).astype(jnp.float32)

    x2d = x.reshape(N, C, HW)

    flops = N * (18 * C * HW + 2 * C * Co * HW)
    bytes_accessed = 4 * N * HW * (C + Co) + wt.size * 4 + wpt.size * 2

    out = pl.pallas_call(
        lambda *refs: _dsconv_kernel(*refs, W=W, HW=HW),
        out_shape=jax.ShapeDtypeStruct((N, Co, HW), jnp.float32),
        grid=(N,),
        in_specs=[
            pl.BlockSpec((1, C, HW), lambda n: (n, 0, 0)),
            pl.BlockSpec((C, 9), lambda n: (0, 0)),
            pl.BlockSpec((C, 1), lambda n: (0, 0)),
            pl.BlockSpec((Co, C), lambda n: (0, 0)),
            pl.BlockSpec((Co, 1), lambda n: (0, 0)),
        ],
        out_specs=pl.BlockSpec((1, Co, HW), lambda n: (n, 0, 0)),
        compiler_params=pltpu.CompilerParams(
            dimension_semantics=("parallel",),
            vmem_limit_bytes=64 * 1024 * 1024),
        cost_estimate=pl.CostEstimate(flops=int(flops), transcendentals=0,
                                      bytes_accessed=int(bytes_accessed)),
    )(x2d, wt, sdw, wpt, spw)

    return out.reshape(N, Co, H, W)


# speedup vs baseline: 1.6900x; 1.2649x over previous
import jax
import jax.numpy as jnp
from jax import lax
from jax.experimental import pallas as pl
from jax.experimental.pallas import tpu as pltpu


def _dsconv_kernel(x_ref, wt_ref, sdw_ref, wpt_ref, spw_ref, o_ref, *, W, HW):
    x4 = x_ref[0]
    C = x4.shape[0]
    x = x4.reshape(C, HW)

    lane = lax.broadcasted_iota(jnp.int32, (1, HW), 1)
    wi = lax.rem(lane, W)
    mask_w = (wi != 0).astype(jnp.float32)
    mask_e = (wi != W - 1).astype(jnp.float32)
    mask_n = (lane >= W).astype(jnp.float32)
    mask_s = (lane < HW - W).astype(jnp.float32)

    s_west = pltpu.roll(x, 1, axis=1) * mask_w
    s_east = pltpu.roll(x, HW - 1, axis=1) * mask_e
    variants = (s_west, x, s_east)

    acc = jnp.zeros_like(x)
    for dy in range(3):
        shift = (1 - dy) * W
        t = jnp.zeros_like(x)
        for dx in range(3):
            v = variants[dx]
            if shift:
                v = pltpu.roll(v, shift % HW, axis=1)
            t = t + v * wt_ref[:, 3 * dy + dx][:, None]
        if dy == 0:
            t = t * mask_n
        elif dy == 2:
            t = t * mask_s
        acc = acc + t
    dw = jnp.maximum(acc + sdw_ref[:, 0][:, None], 0.0)

    pw = jnp.dot(wpt_ref[...], dw.astype(wpt_ref.dtype),
                 preferred_element_type=jnp.float32)
    pw = jnp.maximum(pw + spw_ref[:, 0][:, None], 0.0)
    o_ref[0] = pw.reshape(pw.shape[0], HW // W, W)


def kernel(x, w_dw, s_dw, w_pw, s_pw):
    N, C, H, W = x.shape
    HW = H * W
    Co = w_pw.shape[1] // W

    taps = w_dw[:, :, C:2 * C]
    wt = jnp.transpose(taps, (2, 0, 1)).reshape(C, 9).astype(jnp.float32)
    sdw = s_dw[0, :C].reshape(C, 1).astype(jnp.float32)
    wpt = jnp.transpose(w_pw[:C, :Co]).astype(jnp.bfloat16)
    spw = s_pw[0, :Co].reshape(Co, 1).astype(jnp.float32)

    flops = N * (18 * C * HW + 2 * C * Co * HW)
    bytes_accessed = 4 * N * HW * (C + Co) + wt.size * 4 + wpt.size * 2

    out = pl.pallas_call(
        lambda *refs: _dsconv_kernel(*refs, W=W, HW=HW),
        out_shape=jax.ShapeDtypeStruct((N, Co, H, W), jnp.float32),
        grid=(N,),
        in_specs=[
            pl.BlockSpec((1, C, H, W), lambda n: (n, 0, 0, 0)),
            pl.BlockSpec((C, 9), lambda n: (0, 0)),
            pl.BlockSpec((C, 1), lambda n: (0, 0)),
            pl.BlockSpec((Co, C), lambda n: (0, 0)),
            pl.BlockSpec((Co, 1), lambda n: (0, 0)),
        ],
        out_specs=pl.BlockSpec((1, Co, H, W), lambda n: (n, 0, 0, 0)),
        compiler_params=pltpu.CompilerParams(
            dimension_semantics=("parallel",),
            vmem_limit_bytes=64 * 1024 * 1024),
        cost_estimate=pl.CostEstimate(flops=int(flops), transcendentals=0,
                                      bytes_accessed=int(bytes_accessed)),
    )(x, wt, sdw, wpt, spw)

    return out
